# baseline (speedup 1.0000x reference)
import functools

import numpy as np

import jax
import jax.numpy as jnp
from jax import lax
from jax.experimental import pallas as pl
from jax.experimental.pallas import tpu as pltpu
from jax.experimental.pallas import tpu_sc as plsc

BS = 2
N = 10000
SEQ = 32
CH = 128
CHW = CH // 2
RW = BS * CHW
D = BS * CH
NL = 16

NW = 32
NPW = 320
NPAD = NW * NPW
K = 4
ROWS = K * SEQ
NCHUNK = NPW // K
NPAIR = NCHUNK // 2
G = 64
CPG = G // K
NSUP = NPW // G

_mesh = plsc.VectorSubcoreMesh(core_axis_name="c", subcore_axis_name="s")


@functools.partial(
    pl.kernel,
    out_type=jax.ShapeDtypeStruct((NPAD, D), jnp.float32),
    mesh=_mesh,
    scratch_types=[
        pltpu.VMEM((NPW * SEQ,), jnp.int32),
        pltpu.VMEM((NPW, SEQ), jnp.float32),
        pltpu.VMEM((2 * ROWS, RW), jnp.int32),
        pltpu.VMEM((2 * 2 * K, D), jnp.float32),
        pltpu.SemaphoreType.DMA,
        pltpu.SemaphoreType.DMA,
    ],
)
def _spiral_sc(xt, idxf, wf, out, idx_v, w_v, rows_v, out_v,
               gsem, osem):
    cid = lax.axis_index("c")
    sid = lax.axis_index("s")
    wid = sid
    nbase = wid * NPW
    ibase = nbase * SEQ

    @pl.when(cid == 0)
    def _core0():
        _body(xt, idxf, wf, out, idx_v, w_v, rows_v, out_v, gsem, osem,
              sid * NPW, sid * NPW * SEQ)
        _body(xt, idxf, wf, out, idx_v, w_v, rows_v, out_v, gsem, osem,
              (16 + sid) * NPW, (16 + sid) * NPW * SEQ)


def _body(xt, idxf, wf, out, idx_v, w_v, rows_v, out_v, gsem, osem,
          nbase, ibase):
    pltpu.sync_copy(idxf.at[pl.ds(ibase, NPW * SEQ)], idx_v)
    pltpu.sync_copy(wf.at[pl.ds(nbase, NPW)], w_v)

    def fire(ci, buf):
        pltpu.async_copy(
            xt.at[idx_v.at[pl.ds(ci * ROWS, ROWS)]],
            rows_v.at[pl.ds(buf * ROWS, ROWS)],
            gsem,
        )

    fire(0, 0)

    def do_chunk(ci, buf, pbuf, half):
        pltpu.make_async_copy(
            xt.at[idx_v.at[pl.ds(0, ROWS)]],
            rows_v.at[pl.ds(buf * ROWS, ROWS)],
            gsem,
        ).wait()

        @pl.when(ci + 1 < NCHUNK)
        def _():
            fire(ci + 1, 1 - buf)

        def node(k, carry):
            r0 = buf * ROWS + k * SEQ
            g = ci * K + k
            accs = [jnp.zeros((NL,), jnp.float32) for _ in range(D // NL)]
            wrow = [w_v[g, pl.ds(h * NL, NL)]
                    for h in range(SEQ // NL)]
            for s in range(SEQ):
                wsc = wrow[s // NL][s % NL]
                for c in range(RW // NL):
                    v = rows_v[r0 + s, pl.ds(c * NL, NL)]
                    lo = lax.bitcast_convert_type(v << 16, jnp.float32)
                    hi = lax.bitcast_convert_type(
                        v & jnp.int32(-65536), jnp.float32)
                    accs[2 * c] = accs[2 * c] + wsc * lo
                    accs[2 * c + 1] = accs[2 * c + 1] + wsc * hi
            orow = pbuf * 2 * K + half * K + k
            for c in range(D // NL):
                out_v[orow, pl.ds(c * NL, NL)] = accs[c]
            return carry

        lax.fori_loop(0, K, node, 0)

    def do_pair(p, pbuf):
        @pl.when(p >= 2)
        def _():
            pltpu.make_async_copy(
                out_v.at[pl.ds(pbuf * 2 * K, 2 * K)],
                out.at[pl.ds(nbase, 2 * K)],
                osem,
            ).wait()

        do_chunk(2 * p, 0, pbuf, 0)
        do_chunk(2 * p + 1, 1, pbuf, 1)

        pltpu.async_copy(
            out_v.at[pl.ds(pbuf * 2 * K, 2 * K)],
            out.at[pl.ds(nbase + p * 2 * K, 2 * K)],
            osem,
        )

    def outer(i, carry):
        do_pair(2 * i, 0)
        do_pair(2 * i + 1, 1)
        return carry

    lax.fori_loop(0, NPAIR // 2, outer, 0)

    for b in range(2):
        pltpu.make_async_copy(
            out_v.at[pl.ds(b * 2 * K, 2 * K)],
            out.at[pl.ds(nbase, 2 * K)],
            osem,
        ).wait()


VB = 1000


def _gate_tc(x_ref, wg_ref, bg_ref, ws_ref, o_ref):
    xb = x_ref[0]
    gate = lax.dot_general(
        xb, wg_ref[...], (((1,), (1,)), ((), ())),
        preferred_element_type=jnp.float32,
    ) + bg_ref[...]
    o_ref[0] = gate * ws_ref[...]


def kernel(x, indices, W_g, b_g, weight):
    pad = NPAD - N
    j = np.arange(CH)
    perm = 32 * (j // 32) + (j % 32) // 2 + 16 * (j % 2)
    xbf = x.astype(jnp.bfloat16)[:, :, perm].reshape(BS, N, CHW, 2)
    xi32 = jax.lax.bitcast_convert_type(xbf, jnp.int32)
    xt = jnp.pad(
        xi32.transpose(1, 0, 2).reshape(N, RW), ((0, pad), (0, 0)))
    idxf = jnp.pad(indices, ((0, pad), (0, 0))).reshape(-1)
    wp = jnp.pad(weight, ((0, pad), (0, 0)))

    ws_t = _spiral_sc(xt, idxf, wp)

    out = pl.pallas_call(
        _gate_tc,
        grid=(BS, N // VB),
        in_specs=[
            pl.BlockSpec((1, VB, CH), lambda b, i: (b, i, 0)),
            pl.BlockSpec((CH, CH), lambda b, i: (0, 0)),
            pl.BlockSpec((1, CH), lambda b, i: (0, 0)),
            pl.BlockSpec((VB, CH), lambda b, i: (i, b)),
        ],
        out_specs=pl.BlockSpec((1, VB, CH), lambda b, i: (b, i, 0)),
        out_shape=jax.ShapeDtypeStruct((BS, N, CH), jnp.float32),
    )(x, W_g, b_g.reshape(1, CH), ws_t)
    return out


# speedup vs baseline: 2.4312x; 2.4312x over previous
import functools

import jax
import jax.numpy as jnp
from jax import lax
from jax.experimental import pallas as pl
from jax.experimental.pallas import tpu as pltpu
from jax.experimental.pallas import tpu_sc as plsc

BS = 2
N = 10000
SEQ = 32
CH = 128
NL = 16

NT = 16
NPT = 640
NPAD = NT * NPT
NSTAGE = NPAD // NT
NPP = NPT // 2
K = 2
ROWS = K * SEQ
NCHUNK = NPP // K
QN = 4 * K
NQUAD = NCHUNK // 4

_mesh = plsc.VectorSubcoreMesh(core_axis_name="c", subcore_axis_name="s")


@functools.partial(
    pl.kernel,
    out_type=jax.ShapeDtypeStruct((NPAD * BS, CH), jnp.float32),
    mesh=_mesh,
    scratch_types=[
        pltpu.VMEM_SHARED((NPAD, CH), jnp.float32),
        pltpu.VMEM((NPP * SEQ,), jnp.int32),
        pltpu.VMEM((NPP // 4, 4 * SEQ), jnp.float32),
        pltpu.VMEM((2 * ROWS, CH), jnp.float32),
        pltpu.VMEM((2 * QN, CH), jnp.float32),
        pltpu.SemaphoreType.DMA,
        pltpu.SemaphoreType.DMA,
    ],
)
def _spiral_sc(xf, idxf, wf, out, xsh, idx_v, w_v, rows_v, out_v,
               gsem, osem):
    cid = lax.axis_index("c")
    sid = lax.axis_index("s")

    pltpu.sync_copy(
        xf.at[pl.ds(cid * NPAD + sid * NSTAGE, NSTAGE)],
        xsh.at[pl.ds(sid * NSTAGE, NSTAGE)],
    )
    plsc.subcore_barrier()
    for half_pass in range(2):
        wseg = half_pass * NT + sid
        _run_pass(xsh, idxf, wf, out, idx_v, w_v, rows_v, out_v,
                  gsem, osem, cid, wseg * NPP, wseg * (NPP // 4))


def _run_pass(xsh, idxf, wf, out, idx_v, w_v, rows_v, out_v,
              gsem, osem, cid, nbase, wbase):
    pltpu.sync_copy(idxf.at[pl.ds(nbase * SEQ, NPP * SEQ)], idx_v)
    pltpu.sync_copy(wf.at[pl.ds(wbase, NPP // 4)], w_v)

    def fire(ci, buf):
        pltpu.async_copy(
            xsh.at[idx_v.at[pl.ds(ci * ROWS, ROWS)]],
            rows_v.at[pl.ds(buf * ROWS, ROWS)],
            gsem,
        )

    fire(0, 0)

    obase = nbase * 2 + cid * QN

    def do_chunk(ci, buf, qbuf, quarter):
        pltpu.make_async_copy(
            xsh.at[idx_v.at[pl.ds(0, ROWS)]],
            rows_v.at[pl.ds(buf * ROWS, ROWS)],
            gsem,
        ).wait()

        @pl.when(ci + 1 < NCHUNK)
        def _():
            fire(ci + 1, 1 - buf)

        def node(k, carry):
            r0 = buf * ROWS + k * SEQ
            g = ci * K + k
            accs = [jnp.zeros((NL,), jnp.float32) for _ in range(CH // NL)]
            wrow = [w_v[g // 4, pl.ds((g % 4) * SEQ + h * NL, NL)]
                    for h in range(SEQ // NL)]
            for s in range(SEQ):
                wsc = wrow[s // NL][s % NL]
                for c in range(CH // NL):
                    accs[c] = accs[c] + wsc * rows_v[r0 + s, pl.ds(c * NL, NL)]
            orow = qbuf * QN + quarter * K + k
            for c in range(CH // NL):
                out_v[orow, pl.ds(c * NL, NL)] = accs[c]
            return carry

        lax.fori_loop(0, K, node, 0)

    def do_quad(q, qbuf):
        @pl.when(q >= 2)
        def _():
            pltpu.make_async_copy(
                out_v.at[pl.ds(qbuf * QN, QN)],
                out.at[pl.ds(obase, QN)],
                osem,
            ).wait()

        for j in range(4):
            do_chunk(4 * q + j, j % 2, qbuf, j)

        pltpu.async_copy(
            out_v.at[pl.ds(qbuf * QN, QN)],
            out.at[pl.ds(obase + q * 2 * QN, QN)],
            osem,
        )

    def outer(i, carry):
        do_quad(2 * i, 0)
        do_quad(2 * i + 1, 1)
        return carry

    lax.fori_loop(0, NQUAD // 2, outer, 0)

    for b in range(2):
        pltpu.make_async_copy(
            out_v.at[pl.ds(b * QN, QN)],
            out.at[pl.ds(obase, QN)],
            osem,
        ).wait()


VB = 1000


def _gate_tc(x_ref, wg_ref, bg_ref, ws_ref, o_ref):
    xb = x_ref[0]
    gate = lax.dot_general(
        xb, wg_ref[...], (((1,), (1,)), ((), ())),
        preferred_element_type=jnp.float32,
    ) + bg_ref[...]
    o_ref[0] = gate * ws_ref[0]


def kernel(x, indices, W_g, b_g, weight):
    pad = NPAD - N
    xf = jnp.pad(x, ((0, 0), (0, pad), (0, 0))).reshape(BS * NPAD, CH)
    idxf = jnp.pad(indices, ((0, pad), (0, 0))).reshape(-1)
    wf = jnp.pad(weight, ((0, pad), (0, 0))).reshape(NPAD // 4, 4 * SEQ)

    ws = _spiral_sc(xf, idxf, wf).reshape(NPAD // 8, BS, 8, CH).transpose(
        1, 0, 2, 3).reshape(BS, NPAD, CH)

    out = pl.pallas_call(
        _gate_tc,
        grid=(BS, N // VB),
        in_specs=[
            pl.BlockSpec((1, VB, CH), lambda b, i: (b, i, 0)),
            pl.BlockSpec((CH, CH), lambda b, i: (0, 0)),
            pl.BlockSpec((1, CH), lambda b, i: (0, 0)),
            pl.BlockSpec((1, VB, CH), lambda b, i: (b, i, 0)),
        ],
        out_specs=pl.BlockSpec((1, VB, CH), lambda b, i: (b, i, 0)),
        out_shape=jax.ShapeDtypeStruct((BS, N, CH), jnp.float32),
    )(x, W_g, b_g.reshape(1, CH), ws)
    return out


# speedup vs baseline: 2.6961x; 1.1089x over previous
import functools

import jax
import jax.numpy as jnp
from jax import lax
from jax.experimental import pallas as pl
from jax.experimental.pallas import tpu as pltpu
from jax.experimental.pallas import tpu_sc as plsc

BS = 2
N = 10000
SEQ = 32
CH = 128
NL = 16

NT = 16
NPT = 640
NPAD = NT * NPT
NSTAGE = NPAD // NT
NPP = NPT // 2
K = 2
ROWS = K * SEQ
NCHUNK = NPP // K
QN = 4 * K
NQUAD = NCHUNK // 4

_mesh = plsc.VectorSubcoreMesh(core_axis_name="c", subcore_axis_name="s")


@functools.partial(
    pl.kernel,
    out_type=jax.ShapeDtypeStruct((NPAD * BS, CH), jnp.float32),
    mesh=_mesh,
    scratch_types=[
        pltpu.VMEM_SHARED((NPAD, CH), jnp.float32),
        pltpu.VMEM((NPP * SEQ,), jnp.int32),
        pltpu.VMEM((NPP // 4, 4 * SEQ), jnp.float32),
        pltpu.VMEM((2 * ROWS, CH), jnp.float32),
        pltpu.VMEM((2 * QN, CH), jnp.float32),
        pltpu.SemaphoreType.DMA,
        pltpu.SemaphoreType.DMA,
    ],
)
def _spiral_sc(xf, idxf, wf, out, xsh, idx_v, w_v, rows_v, out_v,
               gsem, osem):
    cid = lax.axis_index("c")
    sid = lax.axis_index("s")

    NSHORT = N - 15 * NSTAGE

    @pl.when(sid < 15)
    def _():
        pltpu.sync_copy(
            xf.at[pl.ds(cid * N + sid * NSTAGE, NSTAGE)],
            xsh.at[pl.ds(sid * NSTAGE, NSTAGE)],
        )

    @pl.when(sid == 15)
    def _():
        pltpu.sync_copy(
            xf.at[pl.ds(cid * N + 15 * NSTAGE, NSHORT)],
            xsh.at[pl.ds(15 * NSTAGE, NSHORT)],
        )

    plsc.subcore_barrier()
    for half_pass in range(2):
        wseg = half_pass * NT + sid
        _run_pass(xsh, idxf, wf, out, idx_v, w_v, rows_v, out_v,
                  gsem, osem, cid, wseg * NPP, wseg * (NPP // 4))


def _run_pass(xsh, idxf, wf, out, idx_v, w_v, rows_v, out_v,
              gsem, osem, cid, nbase, wbase):
    pltpu.sync_copy(idxf.at[pl.ds(nbase * SEQ, NPP * SEQ)], idx_v)
    pltpu.sync_copy(wf.at[pl.ds(wbase, NPP // 4)], w_v)

    def fire(ci, buf):
        pltpu.async_copy(
            xsh.at[idx_v.at[pl.ds(ci * ROWS, ROWS)]],
            rows_v.at[pl.ds(buf * ROWS, ROWS)],
            gsem,
        )

    fire(0, 0)

    obase = nbase * 2 + cid * QN

    def do_chunk(ci, buf, qbuf, quarter):
        pltpu.make_async_copy(
            xsh.at[idx_v.at[pl.ds(0, ROWS)]],
            rows_v.at[pl.ds(buf * ROWS, ROWS)],
            gsem,
        ).wait()

        @pl.when(ci + 1 < NCHUNK)
        def _():
            fire(ci + 1, 1 - buf)

        def node(k, carry):
            r0 = buf * ROWS + k * SEQ
            g = ci * K + k
            accs = [jnp.zeros((NL,), jnp.float32) for _ in range(CH // NL)]
            wrow = [w_v[g // 4, pl.ds((g % 4) * SEQ + h * NL, NL)]
                    for h in range(SEQ // NL)]
            for s in range(SEQ):
                wsc = wrow[s // NL][s % NL]
                for c in range(CH // NL):
                    accs[c] = accs[c] + wsc * rows_v[r0 + s, pl.ds(c * NL, NL)]
            orow = qbuf * QN + quarter * K + k
            for c in range(CH // NL):
                out_v[orow, pl.ds(c * NL, NL)] = accs[c]
            return carry

        lax.fori_loop(0, K, node, 0)

    def do_quad(q, qbuf):
        @pl.when(q >= 2)
        def _():
            pltpu.make_async_copy(
                out_v.at[pl.ds(qbuf * QN, QN)],
                out.at[pl.ds(obase, QN)],
                osem,
            ).wait()

        for j in range(4):
            do_chunk(4 * q + j, j % 2, qbuf, j)

        pltpu.async_copy(
            out_v.at[pl.ds(qbuf * QN, QN)],
            out.at[pl.ds(obase + q * 2 * QN, QN)],
            osem,
        )

    def outer(i, carry):
        do_quad(2 * i, 0)
        do_quad(2 * i + 1, 1)
        return carry

    lax.fori_loop(0, NQUAD // 2, outer, 0)

    for b in range(2):
        pltpu.make_async_copy(
            out_v.at[pl.ds(b * QN, QN)],
            out.at[pl.ds(obase, QN)],
            osem,
        ).wait()


VB = 1000


def _gate_tc(x_ref, wg_ref, bg_ref, ws_ref, o_ref):
    xb = x_ref[0]
    gate = lax.dot_general(
        xb, wg_ref[...], (((1,), (1,)), ((), ())),
        preferred_element_type=jnp.float32,
    ) + bg_ref[...]
    o_ref[0] = gate * ws_ref[...].reshape(VB, CH)


def kernel(x, indices, W_g, b_g, weight):
    pad = NPAD - N
    xf = x.reshape(BS * N, CH)
    idxf = jnp.pad(indices, ((0, pad), (0, 0))).reshape(-1)
    wf = jnp.pad(weight, ((0, pad), (0, 0))).reshape(NPAD // 4, 4 * SEQ)

    ws4 = _spiral_sc(xf, idxf, wf).reshape(NPAD // 8, BS, 8, CH)

    out = pl.pallas_call(
        _gate_tc,
        grid=(BS, N // VB),
        in_specs=[
            pl.BlockSpec((1, VB, CH), lambda b, i: (b, i, 0)),
            pl.BlockSpec((CH, CH), lambda b, i: (0, 0)),
            pl.BlockSpec((1, CH), lambda b, i: (0, 0)),
            pl.BlockSpec((VB // 8, 1, 8, CH), lambda b, i: (i, b, 0, 0)),
        ],
        out_specs=pl.BlockSpec((1, VB, CH), lambda b, i: (b, i, 0)),
        out_shape=jax.ShapeDtypeStruct((BS, N, CH), jnp.float32),
    )(x, W_g, b_g.reshape(1, CH), ws4)
    return out
